# baseline (speedup 1.0000x reference)
import jax
import jax.numpy as jnp
from jax import lax
from jax.experimental import pallas as pl
from jax.experimental.pallas import tpu as pltpu, tpu_sc as plsc

ITEM_NUM = 1000000
ATTR_NUM = 100000
RATING_NUM = 5
EMBED_DIM = 32
ATTR_FNUM = 2
MAX_HIST_LEN = 200
BATCH = 4096
FIELD_NUM = 7

NC = 2
NS = 16
NW = NC * NS
B_PER_W = BATCH // NW
L = MAX_HIST_LEN
AL = ATTR_FNUM * MAX_HIST_LEN
INV_L = 1.0 / MAX_HIST_LEN

ITEM_CHUNKS = ((0, 104), (104, 96))
ATTR_CHUNKS = ((0, 104), (104, 104), (208, 104), (312, 88))


def _zeros():
    return jnp.zeros((16,), jnp.float32)


def _sc_body(hi_hbm, ha_hbm, hr_hbm, iid_hbm, aid_hbm,
             item_t, attr_t, rating_t, out_hbm,
             outbuf, rt_v, ii_v, av_v,
             item_idx0, item_idx1, attr_idx0, attr_idx1, rate_idx0, rate_idx1,
             item_rows0, item_rows1, attr_rows0, attr_rows1,
             sem_idx0, sem_idx1, sem_rows0, sem_rows1, sem_a):
    item_idx = (item_idx0, item_idx1)
    attr_idx = (attr_idx0, attr_idx1)
    rate_idx = (rate_idx0, rate_idx1)
    item_rows = (item_rows0, item_rows1)
    attr_rows = (attr_rows0, attr_rows1)
    sem_idx = (sem_idx0, sem_idx1)
    sem_rows = (sem_rows0, sem_rows1)

    wid = lax.axis_index("s") * NC + lax.axis_index("c")
    base = wid * B_PER_W

    pltpu.sync_copy(rating_t, rt_v)

    pltpu.sync_copy(iid_hbm.at[wid], ii_v)
    pltpu.sync_copy(aid_hbm.at[wid], av_v)
    pltpu.async_copy(item_t.at[ii_v], item_rows0.at[pl.ds(0, 128)], sem_a)
    for c in range(2):
        pltpu.async_copy(attr_t.at[av_v.at[c]],
                         attr_rows0.at[pl.ds(c * 128, 128)], sem_a)
    pltpu.make_async_copy(item_t.at[pl.ds(0, 128)],
                          item_rows0.at[pl.ds(0, 128)], sem_a).wait()
    for c in range(2):
        pltpu.make_async_copy(attr_t.at[pl.ds(0, 128)],
                              attr_rows0.at[pl.ds(c * 128, 128)], sem_a).wait()

    @pl.loop(0, B_PER_W)
    def _copy_single(i):
        for v in range(2):
            sl = pl.ds(v * 16, 16)
            outbuf[i, 0, sl] = item_rows0[i, sl]
            outbuf[i, 1, sl] = attr_rows0[2 * i, sl]
            outbuf[i, 2, sl] = attr_rows0[2 * i + 1, sl]

    def start_idx(gb, slot):
        pltpu.async_copy(hi_hbm.at[pl.ds(gb * L, L)], item_idx[slot],
                         sem_idx[slot])
        pltpu.async_copy(ha_hbm.at[pl.ds(gb * AL, AL)], attr_idx[slot],
                         sem_idx[slot])
        pltpu.async_copy(hr_hbm.at[pl.ds(gb * L, L)], rate_idx[slot],
                         sem_idx[slot])

    def wait_idx(slot):
        pltpu.make_async_copy(hi_hbm.at[pl.ds(0, L)], item_idx[slot],
                              sem_idx[slot]).wait()
        pltpu.make_async_copy(ha_hbm.at[pl.ds(0, AL)], attr_idx[slot],
                              sem_idx[slot]).wait()
        pltpu.make_async_copy(hr_hbm.at[pl.ds(0, L)], rate_idx[slot],
                              sem_idx[slot]).wait()

    def start_gathers(slot):
        for off, ln in ITEM_CHUNKS:
            pltpu.async_copy(item_t.at[item_idx[slot].at[pl.ds(off, ln)]],
                             item_rows[slot].at[pl.ds(off, ln)],
                             sem_rows[slot])
        for off, ln in ATTR_CHUNKS:
            pltpu.async_copy(attr_t.at[attr_idx[slot].at[pl.ds(off, ln)]],
                             attr_rows[slot].at[pl.ds(off, ln)],
                             sem_rows[slot])

    def wait_gathers(slot):
        for off, ln in ITEM_CHUNKS:
            pltpu.make_async_copy(item_t.at[pl.ds(0, ln)],
                                  item_rows[slot].at[pl.ds(off, ln)],
                                  sem_rows[slot]).wait()
        for off, ln in ATTR_CHUNKS:
            pltpu.make_async_copy(attr_t.at[pl.ds(0, ln)],
                                  attr_rows[slot].at[pl.ds(off, ln)],
                                  sem_rows[slot]).wait()

    def rating(k, slot):
        counts = [jnp.zeros((16,), jnp.int32) for _ in range(RATING_NUM)]
        one = jnp.ones((16,), jnp.int32)
        nil = jnp.zeros((16,), jnp.int32)
        lane = lax.broadcasted_iota(jnp.int32, (16,), 0)
        five = jnp.full((16,), RATING_NUM, jnp.int32)
        for i in range(13):
            if i < 12:
                rv = rate_idx[slot][pl.ds(i * 16, 16)]
            else:
                rv = rate_idx[slot][pl.ds(L - 16, 16)]
                rv = jnp.where(lane >= 8, rv, five)
            for r in range(RATING_NUM):
                counts[r] = counts[r] + jnp.where(rv == r, one, nil)
        acc = [_zeros(), _zeros()]
        for r in range(RATING_NUM):
            tot = counts[r]
            for sh in (8, 4, 2, 1):
                tot = tot + jnp.take_along_axis(tot, lane ^ sh, axis=0)
            w = tot.astype(jnp.float32) * INV_L
            for v in range(2):
                acc[v] += w * rt_v[r, pl.ds(v * 16, 16)]
        for v in range(2):
            outbuf[k, 6, pl.ds(v * 16, 16)] = acc[v]

    def reduce(k, slot):
        ir = item_rows[slot]
        ar = attr_rows[slot]

        def body(l, accs):
            i0, i1, a00, a01, a10, a11 = accs
            s0, s1 = pl.ds(0, 16), pl.ds(16, 16)
            i0 = i0 + ir[l, s0]
            i1 = i1 + ir[l, s1]
            a00 = a00 + ar[2 * l, s0]
            a01 = a01 + ar[2 * l, s1]
            a10 = a10 + ar[2 * l + 1, s0]
            a11 = a11 + ar[2 * l + 1, s1]
            return i0, i1, a00, a01, a10, a11

        init = (_zeros(), _zeros(), _zeros(), _zeros(), _zeros(), _zeros())
        i0, i1, a00, a01, a10, a11 = lax.fori_loop(
            0, MAX_HIST_LEN, body, init, unroll=4)
        s0, s1 = pl.ds(0, 16), pl.ds(16, 16)
        outbuf[k, 3, s0] = i0 * INV_L
        outbuf[k, 3, s1] = i1 * INV_L
        outbuf[k, 4, s0] = a00 * INV_L
        outbuf[k, 4, s1] = a01 * INV_L
        outbuf[k, 5, s0] = a10 * INV_L
        outbuf[k, 5, s1] = a11 * INV_L

    def step(k, slot, do_idx, do_gather):
        wait_gathers(slot)
        rating(k, slot)
        if do_idx:
            start_idx(base + k + 2, slot)
        if do_gather:
            wait_idx(1 - slot)
            start_gathers(1 - slot)
        reduce(k, slot)

    start_idx(base + 0, 0)
    start_idx(base + 1, 1)
    wait_idx(0)
    start_gathers(0)

    @pl.loop(0, B_PER_W - 4, step=2)
    def _main(k):
        step(k, 0, True, True)
        step(k + 1, 1, True, True)

    step(B_PER_W - 4, 0, True, True)
    step(B_PER_W - 3, 1, True, True)
    step(B_PER_W - 2, 0, False, True)
    step(B_PER_W - 1, 1, False, False)

    pltpu.sync_copy(outbuf, out_hbm.at[pl.ds(base, B_PER_W)])


@jax.jit
def _run(hi_f, ha_f, hr_f, iid2, aid3, item_table, attr_table, rating_table):
    mesh = plsc.VectorSubcoreMesh(core_axis_name="c", subcore_axis_name="s")
    f = pl.kernel(
        _sc_body,
        out_type=jax.ShapeDtypeStruct((BATCH, FIELD_NUM, EMBED_DIM),
                                      jnp.float32),
        mesh=mesh,
        scratch_types=[
            pltpu.VMEM((B_PER_W, FIELD_NUM, EMBED_DIM), jnp.float32),
            pltpu.VMEM((RATING_NUM + 1, EMBED_DIM), jnp.float32),
            pltpu.VMEM((B_PER_W,), jnp.int32),
            pltpu.VMEM((2, 128), jnp.int32),
            pltpu.VMEM((L,), jnp.int32),
            pltpu.VMEM((L,), jnp.int32),
            pltpu.VMEM((AL,), jnp.int32),
            pltpu.VMEM((AL,), jnp.int32),
            pltpu.VMEM((L,), jnp.int32),
            pltpu.VMEM((L,), jnp.int32),
            pltpu.VMEM((L, EMBED_DIM), jnp.float32),
            pltpu.VMEM((L, EMBED_DIM), jnp.float32),
            pltpu.VMEM((AL, EMBED_DIM), jnp.float32),
            pltpu.VMEM((AL, EMBED_DIM), jnp.float32),
            pltpu.SemaphoreType.DMA,
            pltpu.SemaphoreType.DMA,
            pltpu.SemaphoreType.DMA,
            pltpu.SemaphoreType.DMA,
            pltpu.SemaphoreType.DMA,
        ],
        compiler_params=pltpu.CompilerParams(use_tc_tiling_on_sc=False),
    )
    return f(hi_f, ha_f, hr_f, iid2, aid3, item_table, attr_table,
             rating_table)


def kernel(hist_iid_seq, hist_aid_seq, hist_rate_seq, hist_seq_len, iid, aid,
           lb, item_table, attr_table, rating_table):
    del hist_seq_len, lb
    hi_f = hist_iid_seq.astype(jnp.int32).reshape(BATCH * L)
    ha_f = hist_aid_seq.astype(jnp.int32).reshape(BATCH * AL)
    hr_f = hist_rate_seq.astype(jnp.int32).reshape(BATCH * L)
    iid2 = iid.astype(jnp.int32).reshape(NW, B_PER_W)
    aid3 = aid.astype(jnp.int32).reshape(NW, 2, B_PER_W)
    return _run(hi_f, ha_f, hr_f, iid2, aid3,
                item_table.astype(jnp.float32),
                attr_table.astype(jnp.float32),
                rating_table.astype(jnp.float32))


# speedup vs baseline: 1.2512x; 1.2512x over previous
import jax
import jax.numpy as jnp
from jax import lax
from jax.experimental import pallas as pl
from jax.experimental.pallas import tpu as pltpu, tpu_sc as plsc

ITEM_NUM = 1000000
ATTR_NUM = 100000
RATING_NUM = 5
EMBED_DIM = 32
ATTR_FNUM = 2
MAX_HIST_LEN = 200
BATCH = 4096
FIELD_NUM = 7

NC = 2
NS = 16
NW = NC * NS
B_PER_W = BATCH // NW
L = MAX_HIST_LEN
LP = 208
APL = 416
INV_L = 1.0 / MAX_HIST_LEN

ITEM_CHUNKS = ((0, 104), (104, 104))
ATTR_CHUNKS = ((0, 104), (104, 104), (208, 104), (312, 104))


def _zeros():
    return jnp.zeros((16,), jnp.float32)


def _sc_body(hi_hbm, ha_hbm, hr_hbm, iid_hbm, aid_hbm,
             item_t, attr_t, rating_t, out_hbm,
             outbuf, rt_v, ii_v, av_v,
             ii0, ii1, ai0, ai1, ri0, ri1,
             irow0, irow1, arow0, arow1,
             sem_idx0, sem_idx1, sem_rows0, sem_rows1, sem_a):
    item_idx = (ii0, ii1)
    attr_idx = (ai0, ai1)
    rate_idx = (ri0, ri1)
    item_rows = (irow0, irow1)
    attr_rows = (arow0, arow1)
    sem_idx = (sem_idx0, sem_idx1)
    sem_rows = (sem_rows0, sem_rows1)

    wid = lax.axis_index("s") * NC + lax.axis_index("c")
    base = wid * B_PER_W

    pltpu.sync_copy(rating_t, rt_v)

    pltpu.sync_copy(iid_hbm.at[pl.ds(base, B_PER_W)], ii_v)
    pltpu.sync_copy(aid_hbm.at[wid], av_v)
    pltpu.async_copy(item_t.at[ii_v], irow0.at[pl.ds(0, 128)], sem_a)
    for c in range(2):
        pltpu.async_copy(attr_t.at[av_v.at[c]],
                         arow0.at[pl.ds(c * 128, 128)], sem_a)
    pltpu.make_async_copy(item_t.at[pl.ds(0, 128)],
                          irow0.at[pl.ds(0, 128)], sem_a).wait()
    for c in range(2):
        pltpu.make_async_copy(attr_t.at[pl.ds(0, 128)],
                              arow0.at[pl.ds(c * 128, 128)], sem_a).wait()

    @pl.loop(0, B_PER_W)
    def _copy_single(i):
        for v in range(2):
            sl = pl.ds(v * 16, 16)
            outbuf[i, 0, sl] = irow0[i, sl]
            outbuf[i, 1, sl] = arow0[2 * i, sl]
            outbuf[i, 2, sl] = arow0[2 * i + 1, sl]

    def start_idx(gb, slot):
        pltpu.async_copy(hi_hbm.at[gb], item_idx[slot], sem_idx[slot])
        pltpu.async_copy(ha_hbm.at[gb], attr_idx[slot], sem_idx[slot])
        pltpu.async_copy(hr_hbm.at[gb], rate_idx[slot], sem_idx[slot])

    def wait_idx(slot):
        pltpu.make_async_copy(hi_hbm.at[0], item_idx[slot],
                              sem_idx[slot]).wait()
        pltpu.make_async_copy(ha_hbm.at[0], attr_idx[slot],
                              sem_idx[slot]).wait()
        pltpu.make_async_copy(hr_hbm.at[0], rate_idx[slot],
                              sem_idx[slot]).wait()

    def start_gathers(slot):
        for off, ln in ITEM_CHUNKS:
            pltpu.async_copy(item_t.at[item_idx[slot].at[pl.ds(off, ln)]],
                             item_rows[slot].at[pl.ds(off, ln)],
                             sem_rows[slot])
        for off, ln in ATTR_CHUNKS:
            pltpu.async_copy(attr_t.at[attr_idx[slot].at[pl.ds(off, ln)]],
                             attr_rows[slot].at[pl.ds(off, ln)],
                             sem_rows[slot])

    def wait_gathers(slot):
        for off, ln in ITEM_CHUNKS:
            pltpu.make_async_copy(item_t.at[pl.ds(0, ln)],
                                  item_rows[slot].at[pl.ds(off, ln)],
                                  sem_rows[slot]).wait()
        for off, ln in ATTR_CHUNKS:
            pltpu.make_async_copy(attr_t.at[pl.ds(0, ln)],
                                  attr_rows[slot].at[pl.ds(off, ln)],
                                  sem_rows[slot]).wait()

    def rating(k, slot):
        counts = [jnp.zeros((16,), jnp.int32) for _ in range(RATING_NUM)]
        one = jnp.ones((16,), jnp.int32)
        nil = jnp.zeros((16,), jnp.int32)
        lane = lax.broadcasted_iota(jnp.int32, (16,), 0)
        for i in range(13):
            rv = rate_idx[slot][pl.ds(i * 16, 16)]
            for r in range(RATING_NUM):
                counts[r] = counts[r] + jnp.where(rv == r, one, nil)
        acc = [_zeros(), _zeros()]
        for r in range(RATING_NUM):
            tot = counts[r]
            for sh in (8, 4, 2, 1):
                tot = tot + jnp.take_along_axis(tot, lane ^ sh, axis=0)
            w = tot.astype(jnp.float32) * INV_L
            for v in range(2):
                acc[v] += w * rt_v[r, pl.ds(v * 16, 16)]
        for v in range(2):
            outbuf[k, 6, pl.ds(v * 16, 16)] = acc[v]

    def reduce(k, slot):
        ir = item_rows[slot]
        ar = attr_rows[slot]

        def body(l, accs):
            i0, i1, a00, a01, a10, a11 = accs
            s0, s1 = pl.ds(0, 16), pl.ds(16, 16)
            i0 = i0 + ir[l, s0]
            i1 = i1 + ir[l, s1]
            a00 = a00 + ar[2 * l, s0]
            a01 = a01 + ar[2 * l, s1]
            a10 = a10 + ar[2 * l + 1, s0]
            a11 = a11 + ar[2 * l + 1, s1]
            return i0, i1, a00, a01, a10, a11

        init = (_zeros(), _zeros(), _zeros(), _zeros(), _zeros(), _zeros())
        i0, i1, a00, a01, a10, a11 = lax.fori_loop(
            0, MAX_HIST_LEN, body, init, unroll=4)
        s0, s1 = pl.ds(0, 16), pl.ds(16, 16)
        outbuf[k, 3, s0] = i0 * INV_L
        outbuf[k, 3, s1] = i1 * INV_L
        outbuf[k, 4, s0] = a00 * INV_L
        outbuf[k, 4, s1] = a01 * INV_L
        outbuf[k, 5, s0] = a10 * INV_L
        outbuf[k, 5, s1] = a11 * INV_L

    def step(k, slot, do_idx, do_gather):
        wait_gathers(slot)
        rating(k, slot)
        if do_idx:
            start_idx(base + k + 2, slot)
        if do_gather:
            wait_idx(1 - slot)
            start_gathers(1 - slot)
        reduce(k, slot)

    start_idx(base + 0, 0)
    start_idx(base + 1, 1)
    wait_idx(0)
    start_gathers(0)

    @pl.loop(0, B_PER_W - 4, step=2)
    def _main(k):
        step(k, 0, True, True)
        step(k + 1, 1, True, True)

    step(B_PER_W - 4, 0, True, True)
    step(B_PER_W - 3, 1, True, True)
    step(B_PER_W - 2, 0, False, True)
    step(B_PER_W - 1, 1, False, False)

    pltpu.sync_copy(outbuf, out_hbm.at[pl.ds(base, B_PER_W)])


@jax.jit
def _run(hi_p, ha_p, hr_p, iid_a, aid3, item_table, attr_table, rating_table):
    mesh = plsc.VectorSubcoreMesh(core_axis_name="c", subcore_axis_name="s")
    f = pl.kernel(
        _sc_body,
        out_type=jax.ShapeDtypeStruct((BATCH, FIELD_NUM, EMBED_DIM),
                                      jnp.float32),
        mesh=mesh,
        scratch_types=[
            pltpu.VMEM((B_PER_W, FIELD_NUM, EMBED_DIM), jnp.float32),
            pltpu.VMEM((RATING_NUM + 1, EMBED_DIM), jnp.float32),
            pltpu.VMEM((B_PER_W,), jnp.int32),
            pltpu.VMEM((2, 128), jnp.int32),
            pltpu.VMEM((LP,), jnp.int32),
            pltpu.VMEM((LP,), jnp.int32),
            pltpu.VMEM((APL,), jnp.int32),
            pltpu.VMEM((APL,), jnp.int32),
            pltpu.VMEM((LP,), jnp.int32),
            pltpu.VMEM((LP,), jnp.int32),
            pltpu.VMEM((LP, EMBED_DIM), jnp.float32),
            pltpu.VMEM((LP, EMBED_DIM), jnp.float32),
            pltpu.VMEM((APL, EMBED_DIM), jnp.float32),
            pltpu.VMEM((APL, EMBED_DIM), jnp.float32),
            pltpu.SemaphoreType.DMA,
            pltpu.SemaphoreType.DMA,
            pltpu.SemaphoreType.DMA,
            pltpu.SemaphoreType.DMA,
            pltpu.SemaphoreType.DMA,
        ],
        compiler_params=pltpu.CompilerParams(use_tc_tiling_on_sc=False),
    )
    return f(hi_p, ha_p, hr_p, iid_a, aid3, item_table, attr_table,
             rating_table)


def kernel(hist_iid_seq, hist_aid_seq, hist_rate_seq, hist_seq_len, iid, aid,
           lb, item_table, attr_table, rating_table):
    del hist_seq_len, lb
    hi_p = jnp.pad(hist_iid_seq.astype(jnp.int32), ((0, 0), (0, LP - L)))
    ha = hist_aid_seq.astype(jnp.int32).reshape(BATCH, 2 * L)
    ha_p = jnp.pad(ha, ((0, 0), (0, APL - 2 * L)))
    hr_p = jnp.pad(hist_rate_seq.astype(jnp.int32), ((0, 0), (0, LP - L)),
                   constant_values=RATING_NUM)
    aid3 = aid.astype(jnp.int32).reshape(NW, 2, B_PER_W)
    return _run(hi_p, ha_p, hr_p, iid.astype(jnp.int32), aid3,
                item_table.astype(jnp.float32),
                attr_table.astype(jnp.float32),
                rating_table.astype(jnp.float32))


# speedup vs baseline: 2.2146x; 1.7700x over previous
import jax
import jax.numpy as jnp
from jax import lax
from jax.experimental import pallas as pl
from jax.experimental.pallas import tpu as pltpu, tpu_sc as plsc

ITEM_NUM = 1000000
ATTR_NUM = 100000
RATING_NUM = 5
EMBED_DIM = 32
ATTR_FNUM = 2
MAX_HIST_LEN = 200
BATCH = 4096
FIELD_NUM = 7

NC = 2
NS = 16
NW = NC * NS
B_PER_W = BATCH // NW
L = MAX_HIST_LEN
LP = 256
APL = 512
INV_L = 1.0 / MAX_HIST_LEN

ITEM_CHUNKS = ((0, 104), (104, 96))
ATTR_CHUNKS = ((0, 104), (104, 104), (208, 104), (312, 88))


def _zeros():
    return jnp.zeros((16,), jnp.float32)


def _sc_body(hi_hbm, ha_hbm, hr_hbm, iid_hbm, aid_hbm,
             item_t, attr_t, rating_t, out_hbm,
             outbuf, rt_v, ii_v, av_v,
             ii0, ii1, ai0, ai1, ri0, ri1,
             irow0, irow1, arow0, arow1,
             sem_idx0, sem_idx1, sem_rows0, sem_rows1, sem_a):
    item_idx = (ii0, ii1)
    attr_idx = (ai0, ai1)
    rate_idx = (ri0, ri1)
    item_rows = (irow0, irow1)
    attr_rows = (arow0, arow1)
    sem_idx = (sem_idx0, sem_idx1)
    sem_rows = (sem_rows0, sem_rows1)

    wid = lax.axis_index("s") * NC + lax.axis_index("c")
    base = wid * B_PER_W

    pltpu.sync_copy(rating_t, rt_v)

    pltpu.sync_copy(iid_hbm.at[pl.ds(base, B_PER_W)], ii_v)
    pltpu.sync_copy(aid_hbm.at[wid], av_v)
    pltpu.async_copy(item_t.at[ii_v], irow0.at[pl.ds(0, 128)], sem_a)
    for c in range(2):
        pltpu.async_copy(attr_t.at[av_v.at[c]],
                         arow0.at[pl.ds(c * 128, 128)], sem_a)
    pltpu.make_async_copy(item_t.at[pl.ds(0, 128)],
                          irow0.at[pl.ds(0, 128)], sem_a).wait()
    for c in range(2):
        pltpu.make_async_copy(attr_t.at[pl.ds(0, 128)],
                              arow0.at[pl.ds(c * 128, 128)], sem_a).wait()

    @pl.loop(0, B_PER_W)
    def _copy_single(i):
        for v in range(2):
            sl = pl.ds(v * 16, 16)
            outbuf[i, 0, sl] = irow0[i, sl]
            outbuf[i, 1, sl] = arow0[2 * i, sl]
            outbuf[i, 2, sl] = arow0[2 * i + 1, sl]

    def start_idx(gb, slot):
        pltpu.async_copy(hi_hbm.at[gb], item_idx[slot], sem_idx[slot])
        pltpu.async_copy(ha_hbm.at[gb], attr_idx[slot], sem_idx[slot])
        pltpu.async_copy(hr_hbm.at[gb], rate_idx[slot], sem_idx[slot])

    def wait_idx(slot):
        pltpu.make_async_copy(hi_hbm.at[0], item_idx[slot],
                              sem_idx[slot]).wait()
        pltpu.make_async_copy(ha_hbm.at[0], attr_idx[slot],
                              sem_idx[slot]).wait()
        pltpu.make_async_copy(hr_hbm.at[0], rate_idx[slot],
                              sem_idx[slot]).wait()

    def start_gathers(slot):
        for off, ln in ITEM_CHUNKS:
            pltpu.async_copy(item_t.at[item_idx[slot].at[pl.ds(off, ln)]],
                             item_rows[slot].at[pl.ds(off, ln)],
                             sem_rows[slot])
        for off, ln in ATTR_CHUNKS:
            pltpu.async_copy(attr_t.at[attr_idx[slot].at[pl.ds(off, ln)]],
                             attr_rows[slot].at[pl.ds(off, ln)],
                             sem_rows[slot])

    def wait_gathers(slot):
        for off, ln in ITEM_CHUNKS:
            pltpu.make_async_copy(item_t.at[pl.ds(0, ln)],
                                  item_rows[slot].at[pl.ds(off, ln)],
                                  sem_rows[slot]).wait()
        for off, ln in ATTR_CHUNKS:
            pltpu.make_async_copy(attr_t.at[pl.ds(0, ln)],
                                  attr_rows[slot].at[pl.ds(off, ln)],
                                  sem_rows[slot]).wait()

    def rating(k, slot):
        counts = [jnp.zeros((16,), jnp.int32) for _ in range(RATING_NUM)]
        one = jnp.ones((16,), jnp.int32)
        nil = jnp.zeros((16,), jnp.int32)
        lane = lax.broadcasted_iota(jnp.int32, (16,), 0)
        for i in range(13):
            rv = rate_idx[slot][pl.ds(i * 16, 16)]
            for r in range(RATING_NUM):
                counts[r] = counts[r] + jnp.where(rv == r, one, nil)
        acc = [_zeros(), _zeros()]
        for r in range(RATING_NUM):
            tot = counts[r]
            for sh in (8, 4, 2, 1):
                tot = tot + jnp.take_along_axis(tot, lane ^ sh, axis=0)
            w = tot.astype(jnp.float32) * INV_L
            for v in range(2):
                acc[v] += w * rt_v[r, pl.ds(v * 16, 16)]
        for v in range(2):
            outbuf[k, 6, pl.ds(v * 16, 16)] = acc[v]

    def reduce(k, slot):
        ir = item_rows[slot]
        ar = attr_rows[slot]

        def body(l, accs):
            i0, i1, a00, a01, a10, a11 = accs
            s0, s1 = pl.ds(0, 16), pl.ds(16, 16)
            i0 = i0 + ir[l, s0]
            i1 = i1 + ir[l, s1]
            a00 = a00 + ar[2 * l, s0]
            a01 = a01 + ar[2 * l, s1]
            a10 = a10 + ar[2 * l + 1, s0]
            a11 = a11 + ar[2 * l + 1, s1]
            return i0, i1, a00, a01, a10, a11

        init = (_zeros(), _zeros(), _zeros(), _zeros(), _zeros(), _zeros())
        i0, i1, a00, a01, a10, a11 = lax.fori_loop(
            0, MAX_HIST_LEN, body, init, unroll=4)
        s0, s1 = pl.ds(0, 16), pl.ds(16, 16)
        outbuf[k, 3, s0] = i0 * INV_L
        outbuf[k, 3, s1] = i1 * INV_L
        outbuf[k, 4, s0] = a00 * INV_L
        outbuf[k, 4, s1] = a01 * INV_L
        outbuf[k, 5, s0] = a10 * INV_L
        outbuf[k, 5, s1] = a11 * INV_L

    def step(k, slot, do_idx, do_gather):
        wait_gathers(slot)
        rating(k, slot)
        if do_idx:
            start_idx(base + k + 2, slot)
        if do_gather:
            wait_idx(1 - slot)
            start_gathers(1 - slot)
        reduce(k, slot)

    start_idx(base + 0, 0)
    start_idx(base + 1, 1)
    wait_idx(0)
    start_gathers(0)

    @pl.loop(0, B_PER_W - 4, step=2)
    def _main(k):
        step(k, 0, True, True)
        step(k + 1, 1, True, True)

    step(B_PER_W - 4, 0, True, True)
    step(B_PER_W - 3, 1, True, True)
    step(B_PER_W - 2, 0, False, True)
    step(B_PER_W - 1, 1, False, False)

    pltpu.sync_copy(outbuf, out_hbm.at[pl.ds(base, B_PER_W)])


@jax.jit
def _run(hi_p, ha_p, hr_p, iid_a, aid3, item_table, attr_table, rating_table):
    mesh = plsc.VectorSubcoreMesh(core_axis_name="c", subcore_axis_name="s")
    f = pl.kernel(
        _sc_body,
        out_type=jax.ShapeDtypeStruct((BATCH, FIELD_NUM, EMBED_DIM),
                                      jnp.float32),
        mesh=mesh,
        scratch_types=[
            pltpu.VMEM((B_PER_W, FIELD_NUM, EMBED_DIM), jnp.float32),
            pltpu.VMEM((RATING_NUM + 1, EMBED_DIM), jnp.float32),
            pltpu.VMEM((B_PER_W,), jnp.int32),
            pltpu.VMEM((2, 128), jnp.int32),
            pltpu.VMEM((LP,), jnp.int32),
            pltpu.VMEM((LP,), jnp.int32),
            pltpu.VMEM((APL,), jnp.int32),
            pltpu.VMEM((APL,), jnp.int32),
            pltpu.VMEM((LP,), jnp.int32),
            pltpu.VMEM((LP,), jnp.int32),
            pltpu.VMEM((LP, EMBED_DIM), jnp.float32),
            pltpu.VMEM((LP, EMBED_DIM), jnp.float32),
            pltpu.VMEM((APL, EMBED_DIM), jnp.float32),
            pltpu.VMEM((APL, EMBED_DIM), jnp.float32),
            pltpu.SemaphoreType.DMA,
            pltpu.SemaphoreType.DMA,
            pltpu.SemaphoreType.DMA,
            pltpu.SemaphoreType.DMA,
            pltpu.SemaphoreType.DMA,
        ],
        compiler_params=pltpu.CompilerParams(use_tc_tiling_on_sc=False),
    )
    return f(hi_p, ha_p, hr_p, iid_a, aid3, item_table, attr_table,
             rating_table)


def kernel(hist_iid_seq, hist_aid_seq, hist_rate_seq, hist_seq_len, iid, aid,
           lb, item_table, attr_table, rating_table):
    del hist_seq_len, lb
    hi_p = jnp.pad(hist_iid_seq.astype(jnp.int32), ((0, 0), (0, LP - L)))
    ha = hist_aid_seq.astype(jnp.int32).reshape(BATCH, 2 * L)
    ha_p = jnp.pad(ha, ((0, 0), (0, APL - 2 * L)))
    hr_p = jnp.pad(hist_rate_seq.astype(jnp.int32), ((0, 0), (0, LP - L)),
                   constant_values=RATING_NUM)
    aid3 = aid.astype(jnp.int32).reshape(NW, 2, B_PER_W)
    return _run(hi_p, ha_p, hr_p, iid.astype(jnp.int32), aid3,
                item_table.astype(jnp.float32),
                attr_table.astype(jnp.float32),
                rating_table.astype(jnp.float32))
